# Initial kernel scaffold; baseline (speedup 1.0000x reference)
import jax
import jax.numpy as jnp
from jax.experimental import pallas as pl


def kernel(particles, weights, edges):
    raise NotImplementedError("write your pallas kernel here")


# speedup vs baseline: 8.6559x; 8.6559x over previous
import functools

import jax
import jax.numpy as jnp
from jax import lax
from jax.experimental import pallas as pl
from jax.experimental.pallas import tpu as pltpu
from jax.experimental.pallas import tpu_sc as plsc

_NUM_WORKERS = 32
_CHUNK = 80


def _edge_diff_sc(table, idx_i, idx_j):
    e_total = idx_i.shape[0]
    row = table.shape[1]
    per_w = e_total // _NUM_WORKERS
    n_chunks = per_w // _CHUNK
    mesh = plsc.VectorSubcoreMesh(core_axis_name="c", subcore_axis_name="s")

    @functools.partial(
        pl.kernel,
        mesh=mesh,
        out_type=jax.ShapeDtypeStruct((e_total, row), table.dtype),
        scratch_types=[
            pltpu.VMEM((per_w,), jnp.int32),
            pltpu.VMEM((per_w,), jnp.int32),
            pltpu.VMEM((_CHUNK, row), jnp.float32),
            pltpu.VMEM((_CHUNK, row), jnp.float32),
            pltpu.SemaphoreType.DMA,
            pltpu.SemaphoreType.DMA,
        ],
    )
    def k(table_hbm, ii_hbm, jj_hbm, out_hbm, ii_v, jj_v, a_v, b_v, sem_a, sem_b):
        wid = lax.axis_index("s") * 2 + lax.axis_index("c")
        base = wid * per_w
        pltpu.sync_copy(ii_hbm.at[pl.ds(base, per_w)], ii_v)
        pltpu.sync_copy(jj_hbm.at[pl.ds(base, per_w)], jj_v)

        @pl.loop(0, n_chunks)
        def _(t):
            off = t * _CHUNK
            ca = pltpu.async_copy(
                table_hbm.at[ii_v.at[pl.ds(off, _CHUNK)]], a_v, sem_a)
            cb = pltpu.async_copy(
                table_hbm.at[jj_v.at[pl.ds(off, _CHUNK)]], b_v, sem_b)
            ca.wait()
            cb.wait()

            @pl.loop(0, _CHUNK)
            def _(r):
                for c in range(row // 16):
                    sl = pl.ds(c * 16, 16)
                    a_v[r, sl] = a_v[r, sl] - b_v[r, sl]

            pltpu.sync_copy(a_v, out_hbm.at[pl.ds(base + off, _CHUNK)])

    return k(table, idx_i, idx_j)


def _bcast_body(w_ref, o_ref):
    o_ref[...] = jnp.broadcast_to(w_ref[...], o_ref.shape)


def _tile_row_tc(w0, e_total):
    blk = 3200
    return pl.pallas_call(
        _bcast_body,
        grid=(e_total // blk,),
        in_specs=[pl.BlockSpec((1, w0.shape[1]), lambda i: (0, 0))],
        out_specs=pl.BlockSpec((blk, w0.shape[1]), lambda i: (i, 0)),
        out_shape=jax.ShapeDtypeStruct((e_total, w0.shape[1]), w0.dtype),
    )(w0)


def kernel(particles, weights, edges):
    n, p, d = particles.shape
    e_total = edges.shape[1]
    table = particles.reshape(n, p * d)
    idx = edges.astype(jnp.int32)
    ratios = _edge_diff_sc(table, idx[0], idx[1]).reshape(e_total, p, d)
    rm_weights = _tile_row_tc(weights[0:1, :], e_total)
    return ratios, rm_weights


# speedup vs baseline: 10.0304x; 1.1588x over previous
import functools

import jax
import jax.numpy as jnp
from jax import lax
from jax.experimental import pallas as pl
from jax.experimental.pallas import tpu as pltpu
from jax.experimental.pallas import tpu_sc as plsc

_NUM_WORKERS = 32
_CHUNK = 80


_NBUF = 4


def _edge_diff_sc(table, idx_i, idx_j):
    e_total = idx_i.shape[0]
    row = table.shape[1]
    per_w = e_total // _NUM_WORKERS
    n_chunks = per_w // _CHUNK
    n_pad = -(-n_chunks // _NBUF) * _NBUF
    mesh = plsc.VectorSubcoreMesh(core_axis_name="c", subcore_axis_name="s")

    @functools.partial(
        pl.kernel,
        mesh=mesh,
        out_type=jax.ShapeDtypeStruct((e_total, row), table.dtype),
        scratch_types=[
            pltpu.VMEM((per_w,), jnp.int32),
            pltpu.VMEM((per_w,), jnp.int32),
        ]
        + [pltpu.VMEM((_CHUNK, row), jnp.float32) for _ in range(2 * _NBUF)]
        + [pltpu.SemaphoreType.DMA for _ in range(3 * _NBUF)],
    )
    def k(table_hbm, ii_hbm, jj_hbm, out_hbm, ii_v, jj_v, *bufs_and_sems):
        a_v = bufs_and_sems[:_NBUF]
        b_v = bufs_and_sems[_NBUF:2 * _NBUF]
        sem_a = bufs_and_sems[2 * _NBUF:3 * _NBUF]
        sem_b = bufs_and_sems[3 * _NBUF:4 * _NBUF]
        sem_w = bufs_and_sems[4 * _NBUF:5 * _NBUF]
        wid = lax.axis_index("s") * 2 + lax.axis_index("c")
        base = wid * per_w
        pltpu.sync_copy(ii_hbm.at[pl.ds(base, per_w)], ii_v)
        pltpu.sync_copy(jj_hbm.at[pl.ds(base, per_w)], jj_v)

        def issue_gather(c, slot):
            off = c * _CHUNK
            pltpu.async_copy(
                table_hbm.at[ii_v.at[pl.ds(off, _CHUNK)]], a_v[slot], sem_a[slot])
            pltpu.async_copy(
                table_hbm.at[jj_v.at[pl.ds(off, _CHUNK)]], b_v[slot], sem_b[slot])

        def wait_gather(slot):
            pltpu.make_async_copy(
                table_hbm.at[ii_v.at[pl.ds(0, _CHUNK)]], a_v[slot], sem_a[slot]).wait()
            pltpu.make_async_copy(
                table_hbm.at[jj_v.at[pl.ds(0, _CHUNK)]], b_v[slot], sem_b[slot]).wait()

        def issue_write(c, slot):
            pltpu.async_copy(
                a_v[slot], out_hbm.at[pl.ds(base + c * _CHUNK, _CHUNK)], sem_w[slot])

        def wait_write(slot):
            pltpu.make_async_copy(
                a_v[slot], out_hbm.at[pl.ds(base, _CHUNK)], sem_w[slot]).wait()

        issue_gather(0, 0)
        issue_gather(1, 1)

        @pl.loop(0, n_pad // _NBUF)
        def _(t):
            for u in range(_NBUF):
                c = t * _NBUF + u

                @pl.when(jnp.logical_and(c >= 2, c - 2 < n_chunks))
                def _():
                    wait_write((u + 2) % _NBUF)

                @pl.when(c + 2 < n_chunks)
                def _():
                    issue_gather(c + 2, (u + 2) % _NBUF)

                @pl.when(c < n_chunks)
                def _():
                    wait_gather(u)

                    @pl.loop(0, _CHUNK)
                    def _(r):
                        for cc in range(row // 16):
                            sl = pl.ds(cc * 16, 16)
                            a_v[u][r, sl] = a_v[u][r, sl] - b_v[u][r, sl]

                    issue_write(c, u)

    return k(table, idx_i, idx_j)


def _bcast_body(w_ref, o_ref):
    o_ref[...] = jnp.broadcast_to(w_ref[...], o_ref.shape)


def _tile_row_tc(w0, e_total):
    blk = 3200
    return pl.pallas_call(
        _bcast_body,
        grid=(e_total // blk,),
        in_specs=[pl.BlockSpec((1, w0.shape[1]), lambda i: (0, 0))],
        out_specs=pl.BlockSpec((blk, w0.shape[1]), lambda i: (i, 0)),
        out_shape=jax.ShapeDtypeStruct((e_total, w0.shape[1]), w0.dtype),
    )(w0)


def kernel(particles, weights, edges):
    n, p, d = particles.shape
    e_total = edges.shape[1]
    table = particles.reshape(n, p * d)
    idx = edges.astype(jnp.int32)
    ratios = _edge_diff_sc(table, idx[0], idx[1]).reshape(e_total, p, d)
    rm_weights = _tile_row_tc(weights[0:1, :], e_total)
    return ratios, rm_weights
